# baseline (speedup 1.0000x reference)
import functools

import jax
import jax.numpy as jnp
from jax import lax
from jax.experimental import pallas as pl
from jax.experimental.pallas import tpu as pltpu
from jax.experimental.pallas import tpu_sc as plsc

B = 16384
EMB = 16
NC, NS = 2, 16
NW = NC * NS
BPW = B // NW
BLK = 4096


def _gather_body(c_idx, u_idx, c_tab, u_tab, c_out, u_out,
                 cidx_v, uidx_v, crows_v, urows_v, sem_c, sem_u):
    wid = lax.axis_index("s") * NC + lax.axis_index("c")
    base = wid * BPW
    pltpu.sync_copy(c_idx.at[pl.ds(base, BPW)], cidx_v)
    pltpu.sync_copy(u_idx.at[pl.ds(base, BPW)], uidx_v)
    cp_c = pltpu.async_copy(c_tab.at[cidx_v], crows_v, sem_c)
    cp_u = pltpu.async_copy(u_tab.at[uidx_v], urows_v, sem_u)
    cp_c.wait()
    pltpu.sync_copy(crows_v, c_out.at[pl.ds(base, BPW)])
    cp_u.wait()
    pltpu.sync_copy(urows_v, u_out.at[pl.ds(base, BPW)])


@functools.cache
def _make_gather():
    return pl.kernel(
        _gather_body,
        mesh=plsc.VectorSubcoreMesh(core_axis_name="c", subcore_axis_name="s",
                                    num_cores=NC, num_subcores=NS),
        out_type=(jax.ShapeDtypeStruct((B, EMB), jnp.float32),
                  jax.ShapeDtypeStruct((B, EMB), jnp.float32)),
        scratch_types=[
            pltpu.VMEM((BPW,), jnp.int32),
            pltpu.VMEM((BPW,), jnp.int32),
            pltpu.VMEM((BPW, EMB), jnp.float32),
            pltpu.VMEM((BPW, EMB), jnp.float32),
            pltpu.SemaphoreType.DMA,
            pltpu.SemaphoreType.DMA,
        ],
        compiler_params=pltpu.CompilerParams(use_tc_tiling_on_sc=False),
    )


def _mlp_body(c_ref, u_ref, w0_ref, b0_ref, w1_ref, b1_ref, w2_ref, b2_ref,
              w3_ref, b3_ref, w4_ref, b4_ref, w5_ref, b5_ref, o_ref):
    f32 = jnp.float32
    x = (jnp.dot(c_ref[...], w0_ref[0:EMB, :], preferred_element_type=f32)
         + jnp.dot(u_ref[...], w0_ref[EMB:2 * EMB, :], preferred_element_type=f32)
         + b0_ref[...])
    x = jnp.dot(x, w1_ref[...], preferred_element_type=f32) + b1_ref[...]
    x = jnp.maximum(x, 0.0)
    x = jnp.dot(x, w2_ref[...], preferred_element_type=f32) + b2_ref[...]
    x = jnp.maximum(x, 0.0)
    x = jnp.dot(x, w3_ref[...], preferred_element_type=f32) + b3_ref[...]
    x = jnp.maximum(x, 0.0)
    x = jnp.dot(x, w4_ref[...], preferred_element_type=f32) + b4_ref[...]
    x = jnp.maximum(x, 0.0)
    x = jnp.dot(x, w5_ref[...], preferred_element_type=f32) + b5_ref[...]
    o_ref[...] = jnp.maximum(x, 0.0)


def _mlp(c_rows, u_rows, W0, b0, W1, b1, W2, b2, W3, b3, W4, b4, W5, b5,
         interpret=False):
    def full(shape):
        return pl.BlockSpec(shape, lambda i: (0, 0))

    return pl.pallas_call(
        _mlp_body,
        grid=(B // BLK,),
        in_specs=[
            pl.BlockSpec((BLK, EMB), lambda i: (i, 0)),
            pl.BlockSpec((BLK, EMB), lambda i: (i, 0)),
            full(W0.shape), full(b0.shape),
            full(W1.shape), full(b1.shape),
            full(W2.shape), full(b2.shape),
            full(W3.shape), full(b3.shape),
            full(W4.shape), full(b4.shape),
            full(W5.shape), full(b5.shape),
        ],
        out_specs=pl.BlockSpec((BLK, 1), lambda i: (i, 0)),
        out_shape=jax.ShapeDtypeStruct((B, 1), jnp.float32),
        interpret=interpret,
    )(c_rows, u_rows, W0, b0, W1, b1, W2, b2, W3, b3, W4, b4, W5, b5)


def kernel(c_idx, u_idx, c_table, u_table, W0, b0, W1, b1, W2, b2,
           W3, b3, W4, b4, W5, b5):
    c_rows, u_rows = _make_gather()(c_idx.astype(jnp.int32),
                                    u_idx.astype(jnp.int32), c_table, u_table)
    r2 = lambda b: b.reshape(1, -1)
    return _mlp(c_rows, u_rows, W0, r2(b0), W1, r2(b1), W2, r2(b2),
                W3, r2(b3), W4, r2(b4), W5, r2(b5))
